# Initial kernel scaffold; baseline (speedup 1.0000x reference)
import jax
import jax.numpy as jnp
from jax.experimental import pallas as pl


def kernel(inputs, table):
    raise NotImplementedError("write your pallas kernel here")


# speedup vs baseline: 1.1127x; 1.1127x over previous
import functools

import jax
import jax.numpy as jnp
from jax import lax
from jax.experimental import pallas as pl
from jax.experimental.pallas import tpu as pltpu
from jax.experimental.pallas import tpu_sc as plsc

_NC = 2
_NS = 16
_NW = _NC * _NS

_WDIMS = 32
_CHUNK = 512
_NBUF = 2


def _make_lookup(n_idx: int):
  per_w = n_idx // _NW
  n_chunks = per_w // _CHUNK
  n_pairs = n_chunks // _NBUF
  assert per_w * _NW == n_idx and n_chunks * _CHUNK == per_w
  assert n_pairs * _NBUF == n_chunks

  mesh = plsc.VectorSubcoreMesh(core_axis_name="c", subcore_axis_name="s")

  @functools.partial(
      pl.kernel,
      mesh=mesh,
      out_type=jax.ShapeDtypeStruct((n_idx, _WDIMS), jnp.float32),
      scratch_types=[
          pltpu.VMEM((per_w,), jnp.int32),
          pltpu.VMEM((_NBUF, _CHUNK, _WDIMS), jnp.float32),
          pltpu.SemaphoreType.DMA((_NBUF,)),
      ],
      compiler_params=pltpu.CompilerParams(use_tc_tiling_on_sc=False),
  )
  def lookup(idx_hbm, table_hbm, out_hbm, idx_v, rows_v, gsem):
    wid = lax.axis_index("s") * _NC + lax.axis_index("c")
    base = wid * per_w
    pltpu.sync_copy(idx_hbm.at[pl.ds(base, per_w)], idx_v)

    def gather(j, buf):
      pltpu.async_copy(
          table_hbm.at[idx_v.at[pl.ds(j * _CHUNK, _CHUNK)]],
          rows_v.at[buf],
          gsem.at[buf],
      )

    def wait_and_flush(j, buf):
      pltpu.make_async_copy(
          table_hbm.at[idx_v.at[pl.ds(j * _CHUNK, _CHUNK)]],
          rows_v.at[buf],
          gsem.at[buf],
      ).wait()
      pltpu.sync_copy(
          rows_v.at[buf],
          out_hbm.at[pl.ds(base + j * _CHUNK, _CHUNK)],
      )

    gather(0, 0)

    def pair_body(p, carry):
      j0 = p * _NBUF
      gather(j0 + 1, 1)
      wait_and_flush(j0, 0)

      @pl.when(p + 1 < n_pairs)
      def _():
        gather(j0 + _NBUF, 0)

      wait_and_flush(j0 + 1, 1)
      return carry

    lax.fori_loop(0, n_pairs, pair_body, 0)

  return lookup


def kernel(inputs, table):
  batch, hist = inputs.shape
  n_idx = batch * hist
  idx = inputs.reshape(n_idx).astype(jnp.int32)
  out = _make_lookup(n_idx)(idx, table)
  return out.reshape(batch, hist, table.shape[1])


# speedup vs baseline: 1.1139x; 1.0011x over previous
import functools

import jax
import jax.numpy as jnp
from jax import lax
from jax.experimental import pallas as pl
from jax.experimental.pallas import tpu as pltpu
from jax.experimental.pallas import tpu_sc as plsc

_NC = 2
_NS = 16
_NW = _NC * _NS

_WDIMS = 32
_CHUNK = 640
_NBUF = 4


def _make_lookup(n_idx: int):
  per_w = n_idx // _NW
  n_chunks = per_w // _CHUNK
  n_groups = n_chunks // _NBUF
  assert per_w * _NW == n_idx and n_chunks * _CHUNK == per_w
  assert n_groups * _NBUF == n_chunks and n_chunks >= _NBUF

  mesh = plsc.VectorSubcoreMesh(core_axis_name="c", subcore_axis_name="s")

  @functools.partial(
      pl.kernel,
      mesh=mesh,
      out_type=jax.ShapeDtypeStruct((n_idx, _WDIMS), jnp.float32),
      scratch_types=[
          pltpu.VMEM((per_w,), jnp.int32),
          pltpu.VMEM((_NBUF, _CHUNK, _WDIMS), jnp.float32),
          pltpu.SemaphoreType.DMA((_NBUF,)),
          pltpu.SemaphoreType.DMA((_NBUF,)),
      ],
      compiler_params=pltpu.CompilerParams(use_tc_tiling_on_sc=False),
  )
  def lookup(idx_hbm, table_hbm, out_hbm, idx_v, rows_v, gsem, osem):
    wid = lax.axis_index("s") * _NC + lax.axis_index("c")
    base = wid * per_w
    pltpu.sync_copy(idx_hbm.at[pl.ds(base, per_w)], idx_v)

    def gather_desc(j, buf):
      return pltpu.make_async_copy(
          table_hbm.at[idx_v.at[pl.ds(j * _CHUNK, _CHUNK)]],
          rows_v.at[buf],
          gsem.at[buf],
      )

    def out_desc(j, buf):
      return pltpu.make_async_copy(
          rows_v.at[buf],
          out_hbm.at[pl.ds(base + j * _CHUNK, _CHUNK)],
          osem.at[buf],
      )

    gather_desc(0, 0).start()
    gather_desc(1, 1).start()

    def step(j, b):
      @pl.when(j >= 2)
      def _():
        out_desc(j - 2, (b + 2) % _NBUF).wait()

      @pl.when(j + 2 < n_chunks)
      def _():
        gather_desc(j + 2, (b + 2) % _NBUF).start()

      gather_desc(j, b).wait()
      out_desc(j, b).start()

    def group_body(g, carry):
      j0 = g * _NBUF
      for b in range(_NBUF):
        step(j0 + b, b)
      return carry

    lax.fori_loop(0, n_groups, group_body, 0)
    out_desc(n_chunks - 2, (n_chunks - 2) % _NBUF).wait()
    out_desc(n_chunks - 1, (n_chunks - 1) % _NBUF).wait()

  return lookup


def kernel(inputs, table):
  batch, hist = inputs.shape
  n_idx = batch * hist
  idx = inputs.reshape(n_idx).astype(jnp.int32)
  out = _make_lookup(n_idx)(idx, table)
  return out.reshape(batch, hist, table.shape[1])


# speedup vs baseline: 1.9428x; 1.7441x over previous
import functools

import jax
import jax.numpy as jnp
from jax import lax
from jax.experimental import pallas as pl
from jax.experimental.pallas import tpu as pltpu
from jax.experimental.pallas import tpu_sc as plsc

_NC = 2
_NS = 16
_NW = _NC * _NS

_WDIMS = 32
_CHUNK = 640
_NBUF = 4


def _make_lookup(n_idx: int):
  per_w = n_idx // _NW
  n_chunks = per_w // _CHUNK
  n_groups = n_chunks // _NBUF
  assert per_w * _NW == n_idx and n_chunks * _CHUNK == per_w
  assert n_groups * _NBUF == n_chunks and n_chunks >= _NBUF

  mesh = plsc.VectorSubcoreMesh(core_axis_name="c", subcore_axis_name="s")

  @functools.partial(
      pl.kernel,
      mesh=mesh,
      out_type=jax.ShapeDtypeStruct((n_idx, _WDIMS), jnp.float32),
      scratch_types=[
          pltpu.VMEM((per_w,), jnp.int32),
          pltpu.VMEM((_NBUF, _CHUNK, _WDIMS), jnp.float32),
          pltpu.SemaphoreType.DMA((_NBUF,)),
          pltpu.SemaphoreType.DMA((_NBUF,)),
      ],
      compiler_params=pltpu.CompilerParams(use_tc_tiling_on_sc=False),
  )
  def lookup(idx_hbm, table_hbm, out_hbm, idx_v, rows_v, gsem, osem):
    wid = lax.axis_index("s") * _NC + lax.axis_index("c")
    base = wid * per_w
    pltpu.sync_copy(idx_hbm.at[pl.ds(base, per_w)], idx_v)

    def gather_desc(j, buf):
      return pltpu.make_async_copy(
          table_hbm.at[idx_v.at[pl.ds(j * _CHUNK, _CHUNK)]],
          rows_v.at[buf],
          gsem.at[buf],
      )

    def out_desc(j, buf):
      return pltpu.make_async_copy(
          rows_v.at[buf],
          out_hbm.at[pl.ds(base + j * _CHUNK, _CHUNK)],
          osem.at[buf],
      )

    gather_desc(0, 0).start()
    gather_desc(1, 1).start()

    def step(j, b):
      @pl.when(j >= 2)
      def _():
        out_desc(j - 2, (b + 2) % _NBUF).wait()

      @pl.when(j + 2 < n_chunks)
      def _():
        gather_desc(j + 2, (b + 2) % _NBUF).start()

      gather_desc(j, b).wait()
      out_desc(j, b).start()

    def group_body(g, carry):
      j0 = g * _NBUF
      for b in range(_NBUF):
        step(j0 + b, b)
      return carry

    lax.fori_loop(0, n_groups, group_body, 0)
    out_desc(n_chunks - 2, (n_chunks - 2) % _NBUF).wait()
    out_desc(n_chunks - 1, (n_chunks - 1) % _NBUF).wait()

  return lookup


def kernel(inputs, table):
  batch, hist = inputs.shape
  n_idx = batch * hist
  idx = inputs.T.reshape(n_idx).astype(jnp.int32)
  out = _make_lookup(n_idx)(idx, table)
  return out.reshape(hist, batch, table.shape[1]).transpose(1, 0, 2)


# speedup vs baseline: 1.9441x; 1.0006x over previous
import functools

import jax
import jax.numpy as jnp
from jax import lax
from jax.experimental import pallas as pl
from jax.experimental.pallas import tpu as pltpu
from jax.experimental.pallas import tpu_sc as plsc

_NC = 2
_NS = 16
_NW = _NC * _NS

_WDIMS = 32
_CHUNK = 800
_NBUF = 4


def _make_lookup(n_idx: int):
  per_w = n_idx // _NW
  n_chunks = per_w // _CHUNK
  n_groups = n_chunks // _NBUF
  assert per_w * _NW == n_idx and n_chunks * _CHUNK == per_w
  assert n_groups * _NBUF == n_chunks and n_chunks >= _NBUF

  mesh = plsc.VectorSubcoreMesh(core_axis_name="c", subcore_axis_name="s")

  @functools.partial(
      pl.kernel,
      mesh=mesh,
      out_type=jax.ShapeDtypeStruct((n_idx, _WDIMS), jnp.float32),
      scratch_types=[
          pltpu.VMEM((per_w,), jnp.int32),
          pltpu.VMEM((_NBUF, _CHUNK, _WDIMS), jnp.float32),
          pltpu.SemaphoreType.DMA((_NBUF,)),
          pltpu.SemaphoreType.DMA((_NBUF,)),
      ],
      compiler_params=pltpu.CompilerParams(use_tc_tiling_on_sc=False),
  )
  def lookup(idx_hbm, table_hbm, out_hbm, idx_v, rows_v, gsem, osem):
    wid = lax.axis_index("s") * _NC + lax.axis_index("c")
    base = wid * per_w
    pltpu.sync_copy(idx_hbm.at[pl.ds(base, per_w)], idx_v)

    def gather_desc(j, buf):
      return pltpu.make_async_copy(
          table_hbm.at[idx_v.at[pl.ds(j * _CHUNK, _CHUNK)]],
          rows_v.at[buf],
          gsem.at[buf],
      )

    def out_desc(j, buf):
      return pltpu.make_async_copy(
          rows_v.at[buf],
          out_hbm.at[pl.ds(base + j * _CHUNK, _CHUNK)],
          osem.at[buf],
      )

    gather_desc(0, 0).start()
    gather_desc(1, 1).start()

    def step(j, b):
      @pl.when(j >= 2)
      def _():
        out_desc(j - 2, (b + 2) % _NBUF).wait()

      @pl.when(j + 2 < n_chunks)
      def _():
        gather_desc(j + 2, (b + 2) % _NBUF).start()

      gather_desc(j, b).wait()
      out_desc(j, b).start()

    def group_body(g, carry):
      j0 = g * _NBUF
      for b in range(_NBUF):
        step(j0 + b, b)
      return carry

    lax.fori_loop(0, n_groups, group_body, 0)
    out_desc(n_chunks - 2, (n_chunks - 2) % _NBUF).wait()
    out_desc(n_chunks - 1, (n_chunks - 1) % _NBUF).wait()

  return lookup


def kernel(inputs, table):
  batch, hist = inputs.shape
  n_idx = batch * hist
  idx = inputs.T.reshape(n_idx).astype(jnp.int32)
  out = _make_lookup(n_idx)(idx, table)
  return out.reshape(hist, batch, table.shape[1]).transpose(1, 0, 2)
